# baseline (device time: 15119 ns/iter reference)
import jax
import jax.numpy as jnp
from jax import lax
from jax.experimental import pallas as pl
from jax.experimental.pallas import tpu as pltpu

N_DEV = 4
Q = 512 // N_DEV
NC = 4
H = Q // NC


def kernel(A, B):
    m, _ = A.shape
    _, n = B.shape

    def body(a_ref, b_ref, out_ref, abf_ref, bbf_ref, qsend_ref, rs_ref,
             qbf_ref, ag_ref, send_sems, recv_sems):
        my_pos = lax.axis_index("i")
        right = (my_pos + 1) % N_DEV
        left = (my_pos - 1) % N_DEV
        diag = (my_pos + 2) % N_DEV

        barrier_sem = pltpu.get_barrier_semaphore()
        for t, inc in [(right, 4), (left, 4), (diag, 1)]:
            pl.semaphore_signal(
                barrier_sem, inc=inc,
                device_id=(t,), device_id_type=pl.DeviceIdType.MESH,
            )

        abf_ref[:, :] = a_ref[:, :].astype(jnp.bfloat16)
        bbf_ref[:, :] = b_ref[:, :].astype(jnp.bfloat16)

        def quarter_dot(t):
            return jnp.dot(
                abf_ref[pl.ds(t * Q, Q), :], bbf_ref[:, :],
                preferred_element_type=jnp.float32,
            )

        for dslot, t in [(2, diag), (0, right), (1, left)]:
            qsend_ref[dslot, :, :] = quarter_dot(t).astype(jnp.bfloat16)
        mine = quarter_dot(my_pos)

        def rs_send(dslot, t, h):
            s = 3 * h + dslot
            rdma = pltpu.make_async_remote_copy(
                src_ref=qsend_ref.at[dslot, pl.ds(h * H, H), :],
                dst_ref=rs_ref.at[dslot, pl.ds(h * H, H), :],
                send_sem=send_sems.at[s],
                recv_sem=recv_sems.at[s],
                device_id=(t,),
                device_id_type=pl.DeviceIdType.MESH,
            )
            rdma.start()
            rs_rdmas[s] = rdma

        rs_rdmas = {}
        pl.semaphore_wait(barrier_sem, 8)
        for h in range(NC):
            rs_send(0, right, h)
            rs_send(1, left, h)
        pl.semaphore_wait(barrier_sem, 1)
        for h in range(NC):
            rs_send(2, diag, h)

        ag_rdmas = {}
        for h in range(NC):
            for dslot in (0, 1, 2):
                rs_rdmas[3 * h + dslot].wait_recv()
            rows = pl.ds(h * H, H)
            reduced = (
                mine[h * H:(h + 1) * H, :]
                + rs_ref[0, rows, :].astype(jnp.float32)
                + rs_ref[1, rows, :].astype(jnp.float32)
                + rs_ref[2, rows, :].astype(jnp.float32)
            )
            finished = jnp.maximum(reduced, 0.0)
            out_ref[pl.ds(my_pos * Q + h * H, H), :] = finished
            qbf_ref[rows, :] = finished.astype(jnp.bfloat16)
            for dslot, t in [(2, diag), (0, right), (1, left)]:
                s = 3 * NC + 3 * h + dslot
                rdma = pltpu.make_async_remote_copy(
                    src_ref=qbf_ref.at[rows, :],
                    dst_ref=ag_ref.at[dslot, rows, :],
                    send_sem=send_sems.at[s],
                    recv_sem=recv_sems.at[s],
                    device_id=(t,),
                    device_id_type=pl.DeviceIdType.MESH,
                )
                rdma.start()
                ag_rdmas[s] = rdma

        for h in range(NC):
            for dslot, src in [(0, left), (1, right), (2, diag)]:
                ag_rdmas[3 * NC + 3 * h + dslot].wait_recv()
                out_ref[pl.ds(src * Q + h * H, H), :] = (
                    ag_ref[dslot, pl.ds(h * H, H), :].astype(jnp.float32)
                )

        for rdma in rs_rdmas.values():
            rdma.wait_send()
        for rdma in ag_rdmas.values():
            rdma.wait_send()

    return pl.pallas_call(
        body,
        out_shape=jax.ShapeDtypeStruct((m, n), jnp.float32),
        in_specs=[
            pl.BlockSpec(memory_space=pltpu.VMEM),
            pl.BlockSpec(memory_space=pltpu.VMEM),
        ],
        out_specs=pl.BlockSpec(memory_space=pltpu.VMEM),
        scratch_shapes=[
            pltpu.VMEM((m, m // 2), jnp.bfloat16),
            pltpu.VMEM((m // 2, n), jnp.bfloat16),
            pltpu.VMEM((3, Q, n), jnp.bfloat16),
            pltpu.VMEM((3, Q, n), jnp.bfloat16),
            pltpu.VMEM((Q, n), jnp.bfloat16),
            pltpu.VMEM((3, Q, n), jnp.bfloat16),
            pltpu.SemaphoreType.DMA((6 * NC,)),
            pltpu.SemaphoreType.DMA((6 * NC,)),
        ],
        compiler_params=pltpu.CompilerParams(collective_id=0),
    )(A, B)


# device time: 13742 ns/iter; 1.1002x vs baseline; 1.1002x over previous
import jax
import jax.numpy as jnp
from jax import lax
from jax.experimental import pallas as pl
from jax.experimental.pallas import tpu as pltpu

N_DEV = 4
Q = 512 // N_DEV
NC = 4
H = Q // NC


def kernel(A, B):
    m, _ = A.shape
    _, n = B.shape

    def body(a_ref, b_ref, out_ref, abf_ref, bbf_ref, qsend_ref, rs_ref,
             qbf_ref, ag_ref, send_sems, recv_sems):
        my_pos = lax.axis_index("i")
        right = (my_pos + 1) % N_DEV
        left = (my_pos - 1) % N_DEV
        diag = (my_pos + 2) % N_DEV

        barrier_sem = pltpu.get_barrier_semaphore()
        for t in [right, left, diag]:
            pl.semaphore_signal(
                barrier_sem, inc=1,
                device_id=(t,), device_id_type=pl.DeviceIdType.MESH,
            )

        abf_ref[:, :] = a_ref[:, :].astype(jnp.bfloat16)
        bbf_ref[:, :] = b_ref[:, :].astype(jnp.bfloat16)

        def quarter_dot(t):
            return jnp.dot(
                abf_ref[pl.ds(t * Q, Q), :], bbf_ref[:, :],
                preferred_element_type=jnp.float32,
            )

        for dslot, t in [(2, diag), (0, right), (1, left)]:
            qsend_ref[dslot, :, :] = quarter_dot(t).astype(jnp.bfloat16)
        mine = quarter_dot(my_pos)

        pl.semaphore_wait(barrier_sem, 3)

        rs_rdmas = {}
        for h in range(NC):
            for dslot, t in [(2, diag), (0, right), (1, left)]:
                s = 3 * h + dslot
                rdma = pltpu.make_async_remote_copy(
                    src_ref=qsend_ref.at[dslot, pl.ds(h * H, H), :],
                    dst_ref=rs_ref.at[dslot, pl.ds(h * H, H), :],
                    send_sem=send_sems.at[s],
                    recv_sem=recv_sems.at[s],
                    device_id=(t,),
                    device_id_type=pl.DeviceIdType.MESH,
                )
                rdma.start()
                rs_rdmas[s] = rdma

        ag_rdmas = {}
        for h in range(NC):
            for dslot in (0, 1, 2):
                rs_rdmas[3 * h + dslot].wait_recv()
            rows = pl.ds(h * H, H)
            reduced = (
                mine[h * H:(h + 1) * H, :]
                + rs_ref[0, rows, :].astype(jnp.float32)
                + rs_ref[1, rows, :].astype(jnp.float32)
                + rs_ref[2, rows, :].astype(jnp.float32)
            )
            finished = jnp.maximum(reduced, 0.0)
            out_ref[pl.ds(my_pos * Q + h * H, H), :] = finished
            qbf_ref[rows, :] = finished.astype(jnp.bfloat16)
            for dslot, t in [(2, diag), (0, right), (1, left)]:
                s = 3 * NC + 3 * h + dslot
                rdma = pltpu.make_async_remote_copy(
                    src_ref=qbf_ref.at[rows, :],
                    dst_ref=ag_ref.at[dslot, rows, :],
                    send_sem=send_sems.at[s],
                    recv_sem=recv_sems.at[s],
                    device_id=(t,),
                    device_id_type=pl.DeviceIdType.MESH,
                )
                rdma.start()
                ag_rdmas[s] = rdma

        for h in range(NC):
            for dslot, src in [(0, left), (1, right), (2, diag)]:
                ag_rdmas[3 * NC + 3 * h + dslot].wait_recv()
                out_ref[pl.ds(src * Q + h * H, H), :] = (
                    ag_ref[dslot, pl.ds(h * H, H), :].astype(jnp.float32)
                )

        for rdma in rs_rdmas.values():
            rdma.wait_send()
        for rdma in ag_rdmas.values():
            rdma.wait_send()

    return pl.pallas_call(
        body,
        out_shape=jax.ShapeDtypeStruct((m, n), jnp.float32),
        in_specs=[
            pl.BlockSpec(memory_space=pltpu.VMEM),
            pl.BlockSpec(memory_space=pltpu.VMEM),
        ],
        out_specs=pl.BlockSpec(memory_space=pltpu.VMEM),
        scratch_shapes=[
            pltpu.VMEM((m, m // 2), jnp.bfloat16),
            pltpu.VMEM((m // 2, n), jnp.bfloat16),
            pltpu.VMEM((3, Q, n), jnp.bfloat16),
            pltpu.VMEM((3, Q, n), jnp.bfloat16),
            pltpu.VMEM((Q, n), jnp.bfloat16),
            pltpu.VMEM((3, Q, n), jnp.bfloat16),
            pltpu.SemaphoreType.DMA((6 * NC,)),
            pltpu.SemaphoreType.DMA((6 * NC,)),
        ],
        compiler_params=pltpu.CompilerParams(collective_id=0),
    )(A, B)
